# baseline (speedup 1.0000x reference)
import functools

import jax
import jax.numpy as jnp
from jax import lax
from jax.experimental import pallas as pl
from jax.experimental.pallas import tpu as pltpu
from jax.experimental.pallas import tpu_sc as plsc

B = 16384
D = 64
_NC = 2
_NS = 16
_NW = _NC * _NS
_BPW = B // _NW


def _sc_gather3(table, i, j, neg_j):
    mesh = plsc.VectorSubcoreMesh(core_axis_name="c", subcore_axis_name="s")
    row_out = jax.ShapeDtypeStruct((B, D), jnp.float32)

    @functools.partial(
        pl.kernel,
        mesh=mesh,
        compiler_params=pltpu.CompilerParams(use_tc_tiling_on_sc=False),
        out_type=[row_out, row_out, row_out],
        scratch_types=[
            pltpu.VMEM((_BPW,), jnp.int32),
            pltpu.VMEM((_BPW,), jnp.int32),
            pltpu.VMEM((_BPW,), jnp.int32),
            pltpu.VMEM((_BPW, D), jnp.float32),
            pltpu.VMEM((_BPW, D), jnp.float32),
            pltpu.VMEM((_BPW, D), jnp.float32),
            pltpu.SemaphoreType.DMA,
        ],
    )
    def k(table_hbm, i_hbm, j_hbm, n_hbm, oi_hbm, oj_hbm, on_hbm,
          idx_i, idx_j, idx_n, rows_i, rows_j, rows_n, sem):
        wid = lax.axis_index("s") * _NC + lax.axis_index("c")
        base = wid * _BPW
        pltpu.sync_copy(i_hbm.at[pl.ds(base, _BPW)], idx_i)
        pltpu.sync_copy(j_hbm.at[pl.ds(base, _BPW)], idx_j)
        pltpu.sync_copy(n_hbm.at[pl.ds(base, _BPW)], idx_n)
        ci = pltpu.async_copy(table_hbm.at[idx_i], rows_i, sem)
        cj = pltpu.async_copy(table_hbm.at[idx_j], rows_j, sem)
        cn = pltpu.async_copy(table_hbm.at[idx_n], rows_n, sem)
        ci.wait()
        cj.wait()
        cn.wait()
        pltpu.sync_copy(rows_i, oi_hbm.at[pl.ds(base, _BPW)])
        pltpu.sync_copy(rows_j, oj_hbm.at[pl.ds(base, _BPW)])
        pltpu.sync_copy(rows_n, on_hbm.at[pl.ds(base, _BPW)])

    return k(table, i, j, neg_j)


_TC_BLK = 2048


def _tc_loss(ui, uj, un):
    def body(ui_ref, uj_ref, un_ref, out_ref):
        @pl.when(pl.program_id(0) == 0)
        def _init():
            out_ref[0, 0] = 0.0

        a = ui_ref[...]
        s_pos = jnp.sum(a * uj_ref[...], axis=1)
        s_neg = jnp.sum(a * un_ref[...], axis=1)

        def softplus(x):
            return jnp.maximum(x, 0.0) + jnp.log1p(jnp.exp(-jnp.abs(x)))

        out_ref[0, 0] += jnp.sum(softplus(-s_pos) + softplus(s_neg)) * (1.0 / B)

    out = pl.pallas_call(
        body,
        grid=(B // _TC_BLK,),
        in_specs=[pl.BlockSpec((_TC_BLK, D), lambda g: (g, 0))] * 3,
        out_specs=pl.BlockSpec((1, 1), lambda g: (0, 0), memory_space=pltpu.SMEM),
        out_shape=jax.ShapeDtypeStruct((1, 1), jnp.float32),
    )(ui, uj, un)
    return out[0, 0]


def kernel(table, i, j, neg_j):
    ui, uj, un = _sc_gather3(
        table, i.astype(jnp.int32), j.astype(jnp.int32), neg_j.astype(jnp.int32)
    )
    return _tc_loss(ui, uj, un)


# speedup vs baseline: 1.0080x; 1.0080x over previous
import functools

import jax
import jax.numpy as jnp
from jax import lax
from jax.experimental import pallas as pl
from jax.experimental.pallas import tpu as pltpu
from jax.experimental.pallas import tpu_sc as plsc

B = 16384
D = 64
L = 16
_NC = 2
_NS = 16
_NW = _NC * _NS
_BPW = B // _NW


def _sc_partial_dots(table, i, j, neg_j):
    mesh = plsc.VectorSubcoreMesh(core_axis_name="c", subcore_axis_name="s")
    parts_out = jax.ShapeDtypeStruct((B, L), jnp.float32)

    @functools.partial(
        pl.kernel,
        mesh=mesh,
        compiler_params=pltpu.CompilerParams(use_tc_tiling_on_sc=False),
        out_type=[parts_out, parts_out],
        scratch_types=[
            pltpu.VMEM((_BPW,), jnp.int32),
            pltpu.VMEM((_BPW,), jnp.int32),
            pltpu.VMEM((_BPW,), jnp.int32),
            pltpu.VMEM((_BPW, D), jnp.float32),
            pltpu.VMEM((_BPW, D), jnp.float32),
            pltpu.VMEM((_BPW, D), jnp.float32),
            pltpu.VMEM((_BPW, L), jnp.float32),
            pltpu.VMEM((_BPW, L), jnp.float32),
            pltpu.SemaphoreType.DMA,
        ],
    )
    def k(table_hbm, i_hbm, j_hbm, n_hbm, pp_hbm, pn_hbm,
          idx_i, idx_j, idx_n, rows_i, rows_j, rows_n, pp_v, pn_v, sem):
        wid = lax.axis_index("s") * _NC + lax.axis_index("c")
        base = wid * _BPW
        pltpu.sync_copy(i_hbm.at[pl.ds(base, _BPW)], idx_i)
        pltpu.sync_copy(j_hbm.at[pl.ds(base, _BPW)], idx_j)
        pltpu.sync_copy(n_hbm.at[pl.ds(base, _BPW)], idx_n)
        ci = pltpu.async_copy(table_hbm.at[idx_i], rows_i, sem)
        cj = pltpu.async_copy(table_hbm.at[idx_j], rows_j, sem)
        cn = pltpu.async_copy(table_hbm.at[idx_n], rows_n, sem)
        ci.wait()
        cj.wait()
        cn.wait()

        def row(r, carry):
            a0 = rows_i[r, pl.ds(0, L)]
            a1 = rows_i[r, pl.ds(L, L)]
            a2 = rows_i[r, pl.ds(2 * L, L)]
            a3 = rows_i[r, pl.ds(3 * L, L)]
            p = (a0 * rows_j[r, pl.ds(0, L)]
                 + a1 * rows_j[r, pl.ds(L, L)]
                 + a2 * rows_j[r, pl.ds(2 * L, L)]
                 + a3 * rows_j[r, pl.ds(3 * L, L)])
            n = (a0 * rows_n[r, pl.ds(0, L)]
                 + a1 * rows_n[r, pl.ds(L, L)]
                 + a2 * rows_n[r, pl.ds(2 * L, L)]
                 + a3 * rows_n[r, pl.ds(3 * L, L)])
            pp_v[r, :] = p
            pn_v[r, :] = n
            return carry

        lax.fori_loop(0, _BPW, row, 0)
        pltpu.sync_copy(pp_v, pp_hbm.at[pl.ds(base, _BPW)])
        pltpu.sync_copy(pn_v, pn_hbm.at[pl.ds(base, _BPW)])

    return k(table, i, j, neg_j)


_TC_BLK = 4096


def _tc_loss(pp, pn):
    def body(pp_ref, pn_ref, out_ref):
        @pl.when(pl.program_id(0) == 0)
        def _init():
            out_ref[0, 0] = 0.0

        s_pos = jnp.sum(pp_ref[...], axis=1)
        s_neg = jnp.sum(pn_ref[...], axis=1)

        def softplus(x):
            return jnp.maximum(x, 0.0) + jnp.log1p(jnp.exp(-jnp.abs(x)))

        out_ref[0, 0] += jnp.sum(softplus(-s_pos) + softplus(s_neg)) * (1.0 / B)

    out = pl.pallas_call(
        body,
        grid=(B // _TC_BLK,),
        in_specs=[pl.BlockSpec((_TC_BLK, L), lambda g: (g, 0))] * 2,
        out_specs=pl.BlockSpec((1, 1), lambda g: (0, 0), memory_space=pltpu.SMEM),
        out_shape=jax.ShapeDtypeStruct((1, 1), jnp.float32),
    )(pp, pn)
    return out[0, 0]


def kernel(table, i, j, neg_j):
    pp, pn = _sc_partial_dots(
        table, i.astype(jnp.int32), j.astype(jnp.int32), neg_j.astype(jnp.int32)
    )
    return _tc_loss(pp, pn)


# speedup vs baseline: 1.6589x; 1.6458x over previous
import functools

import jax
import jax.numpy as jnp
from jax import lax
from jax.experimental import pallas as pl
from jax.experimental.pallas import tpu as pltpu
from jax.experimental.pallas import tpu_sc as plsc

B = 16384
D = 64
L = 16
_NC = 2
_NS = 16
_NW = _NC * _NS
_BPW = B // _NW
_CH = 128
_BATCH = 16
_PROW = B * L // 128


def _sc_partial_dots(table, i, j, neg_j):
    mesh = plsc.VectorSubcoreMesh(core_axis_name="c", subcore_axis_name="s")
    parts_out = jax.ShapeDtypeStruct((B * L,), jnp.float32)

    @functools.partial(
        pl.kernel,
        mesh=mesh,
        out_type=[parts_out, parts_out],
        scratch_types=[
            pltpu.VMEM((_BPW,), jnp.int32),
            pltpu.VMEM((_BPW,), jnp.int32),
            pltpu.VMEM((_BPW,), jnp.int32),
            pltpu.VMEM((_CH, D), jnp.float32),
            pltpu.VMEM((_CH, D), jnp.float32),
            pltpu.VMEM((_CH, D), jnp.float32),
            pltpu.VMEM((_BPW * L,), jnp.float32),
            pltpu.VMEM((_BPW * L,), jnp.float32),
            pltpu.SemaphoreType.DMA,
        ],
    )
    def k(table_hbm, i_hbm, j_hbm, n_hbm, pp_hbm, pn_hbm,
          idx_i, idx_j, idx_n, rows_i, rows_j, rows_n, pp_v, pn_v, sem):
        wid = lax.axis_index("s") * _NC + lax.axis_index("c")
        base = wid * _BPW
        pltpu.sync_copy(i_hbm.at[pl.ds(base, _BPW)], idx_i)
        pltpu.sync_copy(j_hbm.at[pl.ds(base, _BPW)], idx_j)
        pltpu.sync_copy(n_hbm.at[pl.ds(base, _BPW)], idx_n)

        for c in range(_BPW // _CH):
            def fetch_batch(bt, carry, c=c):
                r0 = c * _CH + bt * _BATCH
                iv = idx_i[pl.ds(r0, _BATCH)]
                jv = idx_j[pl.ds(r0, _BATCH)]
                nv = idx_n[pl.ds(r0, _BATCH)]
                copies = []
                for q in range(_BATCH):
                    rr = bt * _BATCH + q
                    copies.append(pltpu.async_copy(
                        table_hbm.at[pl.ds(iv[q], 1), :],
                        rows_i.at[pl.ds(rr, 1), :], sem))
                    copies.append(pltpu.async_copy(
                        table_hbm.at[pl.ds(jv[q], 1), :],
                        rows_j.at[pl.ds(rr, 1), :], sem))
                    copies.append(pltpu.async_copy(
                        table_hbm.at[pl.ds(nv[q], 1), :],
                        rows_n.at[pl.ds(rr, 1), :], sem))
                for cp in copies:
                    cp.wait()
                return carry

            lax.fori_loop(0, _CH // _BATCH, fetch_batch, 0)

            def row(r, carry, c=c):
                p = jnp.zeros((L,), jnp.float32)
                n = jnp.zeros((L,), jnp.float32)
                for q in range(D // L):
                    a = rows_i[r, pl.ds(q * L, L)]
                    p = p + a * rows_j[r, pl.ds(q * L, L)]
                    n = n + a * rows_n[r, pl.ds(q * L, L)]
                pp_v[pl.ds((c * _CH + r) * L, L)] = p
                pn_v[pl.ds((c * _CH + r) * L, L)] = n
                return carry

            lax.fori_loop(0, _CH, row, 0)

        pltpu.sync_copy(pp_v, pp_hbm.at[pl.ds(base * L, _BPW * L)])
        pltpu.sync_copy(pn_v, pn_hbm.at[pl.ds(base * L, _BPW * L)])

    return k(table, i, j, neg_j)


_TC_BLK = 256


def _tc_loss(pp, pn):
    def body(pp_ref, pn_ref, out_ref):
        @pl.when(pl.program_id(0) == 0)
        def _init():
            out_ref[0, 0] = 0.0

        r_idx = lax.broadcasted_iota(jnp.int32, (128, 128), 0)
        c_idx = lax.broadcasted_iota(jnp.int32, (128, 128), 1)
        sel = (r_idx // L == c_idx).astype(jnp.float32)
        s_pos = jnp.dot(pp_ref[...], sel, preferred_element_type=jnp.float32)
        s_neg = jnp.dot(pn_ref[...], sel, preferred_element_type=jnp.float32)

        def softplus(x):
            return jnp.maximum(x, 0.0) + jnp.log1p(jnp.exp(-jnp.abs(x)))

        valid = (lax.broadcasted_iota(jnp.int32, (_TC_BLK, 128), 1)
                 < (128 // L)).astype(jnp.float32)
        contrib = valid * (softplus(-s_pos) + softplus(s_neg))
        out_ref[0, 0] += jnp.sum(contrib) * (1.0 / B)

    out = pl.pallas_call(
        body,
        grid=(_PROW // _TC_BLK,),
        in_specs=[pl.BlockSpec((_TC_BLK, 128), lambda g: (g, 0))] * 2,
        out_specs=pl.BlockSpec((1, 1), lambda g: (0, 0), memory_space=pltpu.SMEM),
        out_shape=jax.ShapeDtypeStruct((1, 1), jnp.float32),
    )(pp, pn)
    return out[0, 0]


def kernel(table, i, j, neg_j):
    pp, pn = _sc_partial_dots(
        table, i.astype(jnp.int32), j.astype(jnp.int32), neg_j.astype(jnp.int32)
    )
    return _tc_loss(pp.reshape(_PROW, 128), pn.reshape(_PROW, 128))


# speedup vs baseline: 2.2796x; 1.3741x over previous
import functools

import jax
import jax.numpy as jnp
from jax import lax
from jax.experimental import pallas as pl
from jax.experimental.pallas import tpu as pltpu
from jax.experimental.pallas import tpu_sc as plsc

B = 16384
D = 64
L = 16
_NC = 2
_NS = 16
_NW = _NC * _NS
_BPW = B // _NW
_CH = 128
_BATCH = 16
_PROW = B * L // 128


def _sc_partial_dots(table, i, j, neg_j):
    mesh = plsc.VectorSubcoreMesh(core_axis_name="c", subcore_axis_name="s")
    parts_out = jax.ShapeDtypeStruct((B * L,), jnp.float32)

    @functools.partial(
        pl.kernel,
        mesh=mesh,
        out_type=[parts_out, parts_out],
        scratch_types=[
            pltpu.VMEM((_BPW,), jnp.int32),
            pltpu.VMEM((_BPW,), jnp.int32),
            pltpu.VMEM((_BPW,), jnp.int32),
            pltpu.VMEM((_CH, 1, D), jnp.float32),
            pltpu.VMEM((_CH, 1, D), jnp.float32),
            pltpu.VMEM((_CH, 1, D), jnp.float32),
            pltpu.VMEM((_BPW * L,), jnp.float32),
            pltpu.VMEM((_BPW * L,), jnp.float32),
            pltpu.SemaphoreType.DMA,
        ],
    )
    def k(table_hbm, i_hbm, j_hbm, n_hbm, pp_hbm, pn_hbm,
          idx_i, idx_j, idx_n, rows_i, rows_j, rows_n, pp_v, pn_v, sem):
        wid = lax.axis_index("s") * _NC + lax.axis_index("c")
        base = wid * _BPW
        pltpu.sync_copy(i_hbm.at[pl.ds(base, _BPW)], idx_i)
        pltpu.sync_copy(j_hbm.at[pl.ds(base, _BPW)], idx_j)
        pltpu.sync_copy(n_hbm.at[pl.ds(base, _BPW)], idx_n)

        for c in range(_BPW // _CH):
            def fetch_batch(bt, carry, c=c):
                r0 = c * _CH + bt * _BATCH
                iv = idx_i[pl.ds(r0, _BATCH)]
                jv = idx_j[pl.ds(r0, _BATCH)]
                nv = idx_n[pl.ds(r0, _BATCH)]
                copies = []
                for q in range(_BATCH):
                    rr = bt * _BATCH + q
                    for vec, dst in ((iv, rows_i), (jv, rows_j), (nv, rows_n)):
                        r8 = vec[q] // 8
                        rs = vec[q] % 8
                        copies.append(pltpu.async_copy(
                            table_hbm.at[pl.ds(r8, 1), pl.ds(rs, 1), :],
                            dst.at[pl.ds(rr, 1), :, :], sem))
                for cp in copies:
                    cp.wait()
                return carry

            lax.fori_loop(0, _CH // _BATCH, fetch_batch, 0)

            def row(r, carry, c=c):
                p = jnp.zeros((L,), jnp.float32)
                n = jnp.zeros((L,), jnp.float32)
                for q in range(D // L):
                    a = rows_i[r, 0, pl.ds(q * L, L)]
                    p = p + a * rows_j[r, 0, pl.ds(q * L, L)]
                    n = n + a * rows_n[r, 0, pl.ds(q * L, L)]
                pp_v[pl.ds((c * _CH + r) * L, L)] = p
                pn_v[pl.ds((c * _CH + r) * L, L)] = n
                return carry

            lax.fori_loop(0, _CH, row, 0)

        pltpu.sync_copy(pp_v, pp_hbm.at[pl.ds(base * L, _BPW * L)])
        pltpu.sync_copy(pn_v, pn_hbm.at[pl.ds(base * L, _BPW * L)])

    return k(table, i, j, neg_j)


_TC_BLK = 256


def _tc_loss(pp, pn):
    def body(pp_ref, pn_ref, out_ref):
        @pl.when(pl.program_id(0) == 0)
        def _init():
            out_ref[0, 0] = 0.0

        r_idx = lax.broadcasted_iota(jnp.int32, (128, 128), 0)
        c_idx = lax.broadcasted_iota(jnp.int32, (128, 128), 1)
        sel = (r_idx // L == c_idx).astype(jnp.float32)
        s_pos = jnp.dot(pp_ref[...], sel, preferred_element_type=jnp.float32)
        s_neg = jnp.dot(pn_ref[...], sel, preferred_element_type=jnp.float32)

        def softplus(x):
            return jnp.maximum(x, 0.0) + jnp.log1p(jnp.exp(-jnp.abs(x)))

        valid = (lax.broadcasted_iota(jnp.int32, (_TC_BLK, 128), 1)
                 < (128 // L)).astype(jnp.float32)
        contrib = valid * (softplus(-s_pos) + softplus(s_neg))
        out_ref[0, 0] += jnp.sum(contrib) * (1.0 / B)

    out = pl.pallas_call(
        body,
        grid=(_PROW // _TC_BLK,),
        in_specs=[pl.BlockSpec((_TC_BLK, 128), lambda g: (g, 0))] * 2,
        out_specs=pl.BlockSpec((1, 1), lambda g: (0, 0), memory_space=pltpu.SMEM),
        out_shape=jax.ShapeDtypeStruct((1, 1), jnp.float32),
    )(pp, pn)
    return out[0, 0]


def kernel(table, i, j, neg_j):
    pp, pn = _sc_partial_dots(
        table.reshape(1000000 // 8, 8, D),
        i.astype(jnp.int32), j.astype(jnp.int32), neg_j.astype(jnp.int32)
    )
    return _tc_loss(pp.reshape(_PROW, 128), pn.reshape(_PROW, 128))
